# baseline (speedup 1.0000x reference)
import functools

import jax
import jax.numpy as jnp
from jax import lax
from jax.experimental import pallas as pl
from jax.experimental.pallas import tpu as pltpu
from jax.experimental.pallas import tpu_sc as plsc

_SCALE = 3.1622776601683795

_NUM_WORKERS = 32
_CHUNK = 128


def _emb_call(H, D, B):
    mesh = plsc.VectorSubcoreMesh(core_axis_name="c", subcore_axis_name="s")
    n_tasks = H

    @functools.partial(
        pl.kernel,
        mesh=mesh,
        out_type=jax.ShapeDtypeStruct((H, D // 8, B // _CHUNK, 8, _CHUNK),
                                      jnp.float32),
        scratch_types=(
            [pltpu.VMEM((n_tasks, _CHUNK), jnp.int32)]
            + [pltpu.VMEM((_CHUNK, D), jnp.float32) for _ in range(4)]
            + [pltpu.VMEM((D, _CHUNK), jnp.float32) for _ in range(4)]
            + [pltpu.SemaphoreType.DMA for _ in range(8)]
        ),
        compiler_params=pltpu.CompilerParams(use_tc_tiling_on_sc=False,
                                             needs_layout_passes=False),
    )
    def emb(xt_hbm, table_hbm, out_hbm, idx_v,
            b0, b1, b2, b3, t0, t1, t2, t3, g0, g1, g2, g3, s0, s1, s2, s3):
        bufs = (b0, b1, b2, b3)
        tbufs = (t0, t1, t2, t3)
        gs = (g0, g1, g2, g3)
        ss = (s0, s1, s2, s3)
        wid = lax.axis_index("s") * 2 + lax.axis_index("c")
        pltpu.sync_copy(xt_hbm.at[:, pl.ds(wid * _CHUNK, _CHUNK)], idx_v)

        iota = lax.iota(jnp.int32, 16)
        idx_d = [iota + c * 16 for c in range(D // 16)]

        def gather_start(j, k):
            pltpu.async_copy(table_hbm.at[idx_v.at[j]], bufs[k], gs[k])

        def gather_wait(j, k):
            pltpu.make_async_copy(table_hbm.at[idx_v.at[j]], bufs[k], gs[k]).wait()

        def store_start(j, k):
            for r in range(D // 8):
                pltpu.async_copy(tbufs[k].at[pl.ds(r * 8, 8)],
                                 out_hbm.at[j, r, wid], ss[k])

        def store_wait(k):
            for r in range(D // 8):
                pltpu.make_async_copy(tbufs[k].at[pl.ds(r * 8, 8)],
                                      out_hbm.at[0, r, wid], ss[k]).wait()

        def transpose_scale(k):
            buf, tbuf = bufs[k], tbufs[k]

            def body_b(b, carry):
                bfull = jnp.full((16,), 0, jnp.int32) + b
                for c in range(D // 16):
                    val = buf[b, pl.ds(c * 16, 16)] * _SCALE
                    plsc.store_scatter(tbuf, [idx_d[c], bfull], val)
                return carry

            lax.fori_loop(0, _CHUNK, body_b, 0)

        gather_start(0, 0)
        gather_start(1, 1)
        gather_start(2, 2)
        gather_wait(0, 0)
        transpose_scale(0)
        store_start(0, 0)
        gather_start(3, 3)
        gather_wait(1, 1)
        transpose_scale(1)
        store_start(1, 1)

        def step(jj, carry):
            j0 = 2 + jj * 4
            for t in range(4):
                j = j0 + t
                k = (2 + t) % 4
                kg = t % 4
                store_wait(kg)
                gather_start(j + 2, kg)
                gather_wait(j, k)
                transpose_scale(k)
                store_start(j, k)
            return carry

        lax.fori_loop(0, (n_tasks - 4) // 4, step, 0)

        gather_wait(n_tasks - 2, 2)
        transpose_scale(2)
        store_start(n_tasks - 2, 2)
        gather_wait(n_tasks - 1, 3)
        transpose_scale(3)
        store_start(n_tasks - 1, 3)
        for k in range(4):
            store_wait(k)

    return emb


def kernel(x, table):
    B, H = x.shape
    V, D = table.shape
    assert B == _NUM_WORKERS * _CHUNK
    assert D % 16 == 0 and H % 4 == 0 and H >= 8
    xt = jnp.transpose(x.astype(jnp.int32))
    out5 = _emb_call(H, D, B)(xt, table)
    out = jnp.transpose(
        jnp.reshape(jnp.transpose(out5, (0, 1, 3, 2, 4)), (H, D, B)),
        (2, 0, 1))
    return out


# speedup vs baseline: 1.4086x; 1.4086x over previous
import functools

import jax
import jax.numpy as jnp
from jax import lax
from jax.experimental import pallas as pl
from jax.experimental.pallas import tpu as pltpu
from jax.experimental.pallas import tpu_sc as plsc

_SCALE = 3.1622776601683795

_NUM_WORKERS = 32
_CHUNK = 128
_ROWS_PER_IT = 8


def _emb_call(n_chunks, D, N):
    mesh = plsc.VectorSubcoreMesh(core_axis_name="c", subcore_axis_name="s")

    @functools.partial(
        pl.kernel,
        mesh=mesh,
        out_type=jax.ShapeDtypeStruct((N, D), jnp.float32),
        scratch_types=(
            [pltpu.VMEM((n_chunks, _CHUNK), jnp.int32)]
            + [pltpu.VMEM((_CHUNK, D), jnp.float32) for _ in range(4)]
            + [pltpu.SemaphoreType.DMA for _ in range(8)]
        ),
        compiler_params=pltpu.CompilerParams(use_tc_tiling_on_sc=False),
    )
    def emb(idx_hbm, table_hbm, out_hbm, idx_v,
            b0, b1, b2, b3, g0, g1, g2, g3, s0, s1, s2, s3):
        bufs = (b0, b1, b2, b3)
        gs = (g0, g1, g2, g3)
        ss = (s0, s1, s2, s3)
        wid = lax.axis_index("s") * 2 + lax.axis_index("c")
        crow = wid * n_chunks
        pltpu.sync_copy(idx_hbm.at[pl.ds(crow, n_chunks)], idx_v)

        def gather_start(j, b):
            pltpu.async_copy(table_hbm.at[idx_v.at[j]], bufs[b], gs[b])

        def gather_wait(j, b):
            pltpu.make_async_copy(table_hbm.at[idx_v.at[j]], bufs[b], gs[b]).wait()

        def store_start(j, b):
            pltpu.async_copy(bufs[b], out_hbm.at[pl.ds((crow + j) * _CHUNK, _CHUNK)], ss[b])

        def store_wait(b):
            pltpu.make_async_copy(bufs[b], out_hbm.at[pl.ds(crow * _CHUNK, _CHUNK)], ss[b]).wait()

        def scale(b):
            buf = bufs[b]

            def body(i, carry):
                r0 = i * _ROWS_PER_IT
                for rr in range(_ROWS_PER_IT):
                    for c in range(D // 16):
                        buf[r0 + rr, pl.ds(c * 16, 16)] = (
                            buf[r0 + rr, pl.ds(c * 16, 16)] * _SCALE)
                return carry

            lax.fori_loop(0, _CHUNK // _ROWS_PER_IT, body, 0)

        gather_start(0, 0)
        gather_start(1, 1)
        gather_start(2, 2)
        gather_wait(0, 0)
        scale(0)
        store_start(0, 0)
        gather_start(3, 3)
        gather_wait(1, 1)
        scale(1)
        store_start(1, 1)

        def step(jj, carry):
            j0 = 2 + jj * 4
            for t in range(4):
                j = j0 + t
                b = (2 + t) % 4
                bg = t % 4
                store_wait(bg)
                gather_start(j + 2, bg)
                gather_wait(j, b)
                scale(b)
                store_start(j, b)
            return carry

        lax.fori_loop(0, (n_chunks - 4) // 4, step, 0)

        gather_wait(n_chunks - 2, 2)
        scale(2)
        store_start(n_chunks - 2, 2)
        gather_wait(n_chunks - 1, 3)
        scale(3)
        store_start(n_chunks - 1, 3)
        for b in range(4):
            store_wait(b)

    return emb


def kernel(x, table):
    B, H = x.shape
    V, D = table.shape
    N = B * H
    assert N % (_NUM_WORKERS * _CHUNK) == 0 and D % 16 == 0
    n_chunks = N // (_NUM_WORKERS * _CHUNK)
    assert n_chunks % 4 == 0 and n_chunks >= 8
    idx = x.reshape(_NUM_WORKERS * n_chunks, _CHUNK).astype(jnp.int32)
    out = _emb_call(n_chunks, D, N)(idx, table)
    return out.reshape(B, H, D)
